# baseline (speedup 1.0000x reference)
import jax
import jax.numpy as jnp
from jax import lax
from jax.experimental import pallas as pl
from jax.experimental.pallas import tpu as pltpu
from jax.experimental.pallas import tpu_sc as plsc

B = 8
N = 8192
Q = 16
G = 32
G3 = G * G * G
NW = 32
PPW = N * B // NW
CHUNK = 128
NCHUNK = PPW // CHUNK
SUB = CHUNK // 16


def _sc_body(coef_hbm, pts_hbm, cp_hbm, vox_hbm, out_hbm,
             pbuf, coef_v, idxv, idx0, idx1, idx2,
             gx, gy, gz, gv, rbx, rby, rbz, accv, sem):
    wid = lax.axis_index("s") * 2 + lax.axis_index("c")
    b = wid // 4
    p0 = (wid % 4) * PPW

    pltpu.sync_copy(pts_hbm.at[pl.ds((b * N + p0) * 3, PPW * 3)], pbuf)
    pltpu.sync_copy(coef_hbm.at[b], coef_v)

    base_idx = (b * G3).astype(jnp.int32)
    lane = lax.iota(jnp.int32, 16)
    lane3 = lane * 3

    copies = []
    for c in range(NCHUNK):
        qb = (wid % 4) * 4 + c // 4
        crow = coef_v[qb, :]
        m00 = crow[0]
        m01 = crow[1]
        m02 = crow[2]
        m10 = crow[3]
        m11 = crow[4]
        m12 = crow[5]
        m20 = crow[6]
        m21 = crow[7]
        m22 = crow[8]
        t0 = crow[9]
        t1 = crow[10]
        t2 = crow[11]

        def phase_a(i, _, c=c, m00=m00, m01=m01, m02=m02, m10=m10, m11=m11,
                    m12=m12, m20=m20, m21=m21, m22=m22, t0=t0, t1=t1, t2=t2):
            off = c * CHUNK + i * 16
            p3 = off * 3 + lane3
            vx = plsc.load_gather(pbuf, [p3])
            vy = plsc.load_gather(pbuf, [p3 + 1])
            vz = plsc.load_gather(pbuf, [p3 + 2])
            rx = m00 * vx + m01 * vy + m02 * vz + t0
            ry = m10 * vx + m11 * vy + m12 * vz + t1
            rz = m20 * vx + m21 * vy + m22 * vz + t2

            def vceil(t):
                ti = t.astype(jnp.int32)
                tf = ti.astype(jnp.float32)
                return ti + jnp.where(t > tf, 1, 0).astype(jnp.int32)

            ix = vceil((rx + 0.5) * G - 0.5)
            iy = vceil((ry + 0.5) * G - 0.5)
            iz = vceil((rz + 0.5) * G - 0.5)
            ind = ix * (G * G) + iy * G + iz
            ind = jnp.minimum(jnp.maximum(ind, 0), G3 - 1) + base_idx
            ind3 = ind * 3
            soff = pl.multiple_of(i * 16, 16)
            rbx[c, pl.ds(soff, 16)] = rx
            rby[c, pl.ds(soff, 16)] = ry
            rbz[c, pl.ds(soff, 16)] = rz
            idxv[c, pl.ds(soff, 16)] = ind
            idx0[c, pl.ds(soff, 16)] = ind3
            idx1[c, pl.ds(soff, 16)] = ind3 + 1
            idx2[c, pl.ds(soff, 16)] = ind3 + 2
            return 0

        lax.fori_loop(0, SUB, phase_a, 0)

        copies.append((
            pltpu.async_copy(cp_hbm.at[idx0.at[c]], gx.at[c], sem),
            pltpu.async_copy(cp_hbm.at[idx1.at[c]], gy.at[c], sem),
            pltpu.async_copy(cp_hbm.at[idx2.at[c]], gz.at[c], sem),
            pltpu.async_copy(vox_hbm.at[idxv.at[c]], gv.at[c], sem),
        ))

    acc = jnp.zeros((16,), jnp.float32)
    for c in range(NCHUNK):
        for d in copies[c]:
            d.wait()

        def phase_c(i, acc, c=c):
            soff = pl.multiple_of(i * 16, 16)
            m = 1.0 - gv[c, pl.ds(soff, 16)]
            dx = (rbx[c, pl.ds(soff, 16)] - gx[c, pl.ds(soff, 16)]) * m
            dy = (rby[c, pl.ds(soff, 16)] - gy[c, pl.ds(soff, 16)]) * m
            dz = (rbz[c, pl.ds(soff, 16)] - gz[c, pl.ds(soff, 16)]) * m
            return acc + (dx * dx + dy * dy + dz * dz)

        acc = lax.fori_loop(0, SUB, phase_c, acc)

    accv[...] = acc
    pltpu.sync_copy(accv, out_hbm.at[wid])


def _tc_reduce_body(x_ref, o_ref):
    o_ref[0, 0] = jnp.sum(x_ref[...]) * (1.0 / (B * N))


def kernel(voxel, points, closest_points, quads):
    mid = jnp.mean(points, axis=1)
    qs = quads[..., 1:]
    qs = qs / jnp.linalg.norm(qs, ord=2, axis=2, keepdims=True)
    qs = jnp.concatenate([jnp.ones((B, Q, 1), jnp.float32), qs], axis=-1)
    qs = 0.707 * qs
    w, x, y, z = qs[..., 0], qs[..., 1], qs[..., 2], qs[..., 3]
    M = jnp.stack([
        w * w + x * x - y * y - z * z, 2 * (x * y - w * z), 2 * (x * z + w * y),
        2 * (x * y + w * z), w * w - x * x + y * y - z * z, 2 * (y * z - w * x),
        2 * (x * z - w * y), 2 * (y * z + w * x), w * w - x * x - y * y + z * z,
    ], axis=-1).reshape(B, Q, 3, 3)
    t = -jnp.einsum("bqij,bj->bqi", M, mid)
    coef = jnp.concatenate(
        [M.reshape(B, Q, 9), t, jnp.zeros((B, Q, 4), jnp.float32)], axis=-1)

    pts_flat = points.reshape(B * N * 3)
    cp_flat = closest_points.reshape(B * G3 * 3)
    vox_flat = voxel.reshape(B * G3)

    mesh = plsc.VectorSubcoreMesh(core_axis_name="c", subcore_axis_name="s")
    partials = pl.kernel(
        _sc_body,
        out_type=jax.ShapeDtypeStruct((NW, 16), jnp.float32),
        mesh=mesh,
        compiler_params=pltpu.CompilerParams(
            use_tc_tiling_on_sc=False, needs_layout_passes=False),
        scratch_types=[
            pltpu.VMEM((PPW * 3,), jnp.float32),
            pltpu.VMEM((Q, 16), jnp.float32),
            pltpu.VMEM((NCHUNK, CHUNK), jnp.int32),
            pltpu.VMEM((NCHUNK, CHUNK), jnp.int32),
            pltpu.VMEM((NCHUNK, CHUNK), jnp.int32),
            pltpu.VMEM((NCHUNK, CHUNK), jnp.int32),
            pltpu.VMEM((NCHUNK, CHUNK), jnp.float32),
            pltpu.VMEM((NCHUNK, CHUNK), jnp.float32),
            pltpu.VMEM((NCHUNK, CHUNK), jnp.float32),
            pltpu.VMEM((NCHUNK, CHUNK), jnp.float32),
            pltpu.VMEM((NCHUNK, CHUNK), jnp.float32),
            pltpu.VMEM((NCHUNK, CHUNK), jnp.float32),
            pltpu.VMEM((NCHUNK, CHUNK), jnp.float32),
            pltpu.VMEM((16,), jnp.float32),
            pltpu.SemaphoreType.DMA,
        ],
    )(coef, pts_flat, cp_flat, vox_flat)

    total = pl.pallas_call(
        _tc_reduce_body,
        out_shape=jax.ShapeDtypeStruct((1, 1), jnp.float32),
        out_specs=pl.BlockSpec(memory_space=pltpu.SMEM),
    )(partials)
    return total[0, 0]


# speedup vs baseline: 4.1749x; 4.1749x over previous
import jax
import jax.numpy as jnp
from jax import lax
from jax.experimental import pallas as pl
from jax.experimental.pallas import tpu as pltpu
from jax.experimental.pallas import tpu_sc as plsc

B = 8
N = 8192
Q = 16
G = 32
G3 = G * G * G
NW = 32
PPW = N * B // NW
CHUNK = 128
NCHUNK = PPW // CHUNK
SUB = CHUNK // 16


def _sc_body(coef_hbm, pts_hbm, cp_hbm, vox_hbm, out_hbm,
             px, py, pz, coef_v, idxv, idx0, idx1, idx2,
             gx, gy, gz, gv, rbx, rby, rbz, accv, sem):
    wid = lax.axis_index("s") * 2 + lax.axis_index("c")
    b = wid // 4
    p0 = (wid % 4) * PPW

    pltpu.sync_copy(pts_hbm.at[pl.ds(0 * B * N + b * N + p0, PPW)], px)
    pltpu.sync_copy(pts_hbm.at[pl.ds(1 * B * N + b * N + p0, PPW)], py)
    pltpu.sync_copy(pts_hbm.at[pl.ds(2 * B * N + b * N + p0, PPW)], pz)
    pltpu.sync_copy(coef_hbm.at[b], coef_v)

    base_vox = (b * G3).astype(jnp.int32)
    base_cpx = ((0 * B + b) * G3).astype(jnp.int32)
    base_cpy = ((1 * B + b) * G3).astype(jnp.int32)
    base_cpz = ((2 * B + b) * G3).astype(jnp.int32)

    copies = []
    for c in range(NCHUNK):
        qb = (wid % 4) * 4 + c // 4
        crow = coef_v[qb, :]
        m00 = crow[0]
        m01 = crow[1]
        m02 = crow[2]
        m10 = crow[3]
        m11 = crow[4]
        m12 = crow[5]
        m20 = crow[6]
        m21 = crow[7]
        m22 = crow[8]
        t0 = crow[9]
        t1 = crow[10]
        t2 = crow[11]

        def phase_a(i, _, c=c, m00=m00, m01=m01, m02=m02, m10=m10, m11=m11,
                    m12=m12, m20=m20, m21=m21, m22=m22, t0=t0, t1=t1, t2=t2):
            off = pl.multiple_of(c * CHUNK + i * 16, 16)
            vx = px[pl.ds(off, 16)]
            vy = py[pl.ds(off, 16)]
            vz = pz[pl.ds(off, 16)]
            rx = m00 * vx + m01 * vy + m02 * vz + t0
            ry = m10 * vx + m11 * vy + m12 * vz + t1
            rz = m20 * vx + m21 * vy + m22 * vz + t2

            def vceil(t):
                ti = t.astype(jnp.int32)
                tf = ti.astype(jnp.float32)
                return ti + jnp.where(t > tf, 1, 0).astype(jnp.int32)

            ix = vceil((rx + 0.5) * G - 0.5)
            iy = vceil((ry + 0.5) * G - 0.5)
            iz = vceil((rz + 0.5) * G - 0.5)
            ind = ix * (G * G) + iy * G + iz
            ind = jnp.minimum(jnp.maximum(ind, 0), G3 - 1)
            soff = pl.multiple_of(i * 16, 16)
            rbx[c, pl.ds(soff, 16)] = rx
            rby[c, pl.ds(soff, 16)] = ry
            rbz[c, pl.ds(soff, 16)] = rz
            idxv[c, pl.ds(soff, 16)] = ind + base_vox
            idx0[c, pl.ds(soff, 16)] = ind + base_cpx
            idx1[c, pl.ds(soff, 16)] = ind + base_cpy
            idx2[c, pl.ds(soff, 16)] = ind + base_cpz
            return 0

        lax.fori_loop(0, SUB, phase_a, 0)

        copies.append((
            pltpu.async_copy(cp_hbm.at[idx0.at[c]], gx.at[c], sem),
            pltpu.async_copy(cp_hbm.at[idx1.at[c]], gy.at[c], sem),
            pltpu.async_copy(cp_hbm.at[idx2.at[c]], gz.at[c], sem),
            pltpu.async_copy(vox_hbm.at[idxv.at[c]], gv.at[c], sem),
        ))

    acc = jnp.zeros((16,), jnp.float32)
    for c in range(NCHUNK):
        for d in copies[c]:
            d.wait()

        def phase_c(i, acc, c=c):
            soff = pl.multiple_of(i * 16, 16)
            m = 1.0 - gv[c, pl.ds(soff, 16)]
            dx = (rbx[c, pl.ds(soff, 16)] - gx[c, pl.ds(soff, 16)]) * m
            dy = (rby[c, pl.ds(soff, 16)] - gy[c, pl.ds(soff, 16)]) * m
            dz = (rbz[c, pl.ds(soff, 16)] - gz[c, pl.ds(soff, 16)]) * m
            return acc + (dx * dx + dy * dy + dz * dz)

        acc = lax.fori_loop(0, SUB, phase_c, acc)

    accv[...] = acc
    pltpu.sync_copy(accv, out_hbm.at[wid])


def _tc_reduce_body(x_ref, o_ref):
    o_ref[0, 0] = jnp.sum(x_ref[...]) * (1.0 / (B * N))


def kernel(voxel, points, closest_points, quads):
    mid = jnp.mean(points, axis=1)
    qs = quads[..., 1:]
    qs = qs / jnp.linalg.norm(qs, ord=2, axis=2, keepdims=True)
    qs = jnp.concatenate([jnp.ones((B, Q, 1), jnp.float32), qs], axis=-1)
    qs = 0.707 * qs
    w, x, y, z = qs[..., 0], qs[..., 1], qs[..., 2], qs[..., 3]
    M = jnp.stack([
        w * w + x * x - y * y - z * z, 2 * (x * y - w * z), 2 * (x * z + w * y),
        2 * (x * y + w * z), w * w - x * x + y * y - z * z, 2 * (y * z - w * x),
        2 * (x * z - w * y), 2 * (y * z + w * x), w * w - x * x - y * y + z * z,
    ], axis=-1).reshape(B, Q, 3, 3)
    t = -jnp.einsum("bqij,bj->bqi", M, mid)
    coef = jnp.concatenate(
        [M.reshape(B, Q, 9), t, jnp.zeros((B, Q, 4), jnp.float32)], axis=-1)

    pts_flat = points.transpose(2, 0, 1).reshape(3 * B * N)
    cp_flat = closest_points.transpose(2, 0, 1).reshape(3 * B * G3)
    vox_flat = voxel.reshape(B * G3)

    mesh = plsc.VectorSubcoreMesh(core_axis_name="c", subcore_axis_name="s")
    partials = pl.kernel(
        _sc_body,
        out_type=jax.ShapeDtypeStruct((NW, 16), jnp.float32),
        mesh=mesh,
        compiler_params=pltpu.CompilerParams(
            use_tc_tiling_on_sc=False, needs_layout_passes=False),
        scratch_types=[
            pltpu.VMEM((PPW,), jnp.float32),
            pltpu.VMEM((PPW,), jnp.float32),
            pltpu.VMEM((PPW,), jnp.float32),
            pltpu.VMEM((Q, 16), jnp.float32),
            pltpu.VMEM((NCHUNK, CHUNK), jnp.int32),
            pltpu.VMEM((NCHUNK, CHUNK), jnp.int32),
            pltpu.VMEM((NCHUNK, CHUNK), jnp.int32),
            pltpu.VMEM((NCHUNK, CHUNK), jnp.int32),
            pltpu.VMEM((NCHUNK, CHUNK), jnp.float32),
            pltpu.VMEM((NCHUNK, CHUNK), jnp.float32),
            pltpu.VMEM((NCHUNK, CHUNK), jnp.float32),
            pltpu.VMEM((NCHUNK, CHUNK), jnp.float32),
            pltpu.VMEM((NCHUNK, CHUNK), jnp.float32),
            pltpu.VMEM((NCHUNK, CHUNK), jnp.float32),
            pltpu.VMEM((NCHUNK, CHUNK), jnp.float32),
            pltpu.VMEM((16,), jnp.float32),
            pltpu.SemaphoreType.DMA,
        ],
    )(coef, pts_flat, cp_flat, vox_flat)

    total = pl.pallas_call(
        _tc_reduce_body,
        out_shape=jax.ShapeDtypeStruct((1, 1), jnp.float32),
        out_specs=pl.BlockSpec(memory_space=pltpu.SMEM),
    )(partials)
    return total[0, 0]


# speedup vs baseline: 5.6956x; 1.3643x over previous
import jax
import jax.numpy as jnp
from jax import lax
from jax.experimental import pallas as pl
from jax.experimental.pallas import tpu as pltpu
from jax.experimental.pallas import tpu_sc as plsc

B = 8
N = 8192
Q = 16
G = 32
G3 = G * G * G
NW = 32
PPW = N * B // NW
CHUNK = 128
NCHUNK = PPW // CHUNK
SUB = CHUNK // 16


def _rsqrt(v):
    i = plsc.bitcast(v, jnp.int32)
    i = 0x5F3759DF - (i >> 1)
    y = plsc.bitcast(i, jnp.float32)
    for _ in range(4):
        y = y * (1.5 - 0.5 * v * y * y)
    return y


def _sc_body(pts_hbm, cp_hbm, vox_hbm, quads_hbm, out_hbm,
             px, py, pz, qbuf, coef_v, sumv, idxv, idx0, idx1, idx2,
             gx, gy, gz, gv, rbx, rby, rbz, accv, shared, sem):
    cid = lax.axis_index("c")
    sid = lax.axis_index("s")
    wid = cid * 16 + sid
    b = wid // 4
    p0 = (wid % 4) * PPW

    pltpu.sync_copy(pts_hbm.at[pl.ds(0 * B * N + b * N + p0, PPW)], px)
    pltpu.sync_copy(pts_hbm.at[pl.ds(1 * B * N + b * N + p0, PPW)], py)
    pltpu.sync_copy(pts_hbm.at[pl.ds(2 * B * N + b * N + p0, PPW)], pz)
    pltpu.sync_copy(quads_hbm.at[pl.ds(b * Q * 4, Q * 4)], qbuf)

    lane = lax.iota(jnp.int32, 16)

    def mean_body(i, carry):
        sx, sy, sz = carry
        off = pl.multiple_of(i * 16, 16)
        return (sx + px[pl.ds(off, 16)],
                sy + py[pl.ds(off, 16)],
                sz + pz[pl.ds(off, 16)])

    zero16 = jnp.zeros((16,), jnp.float32)
    sx, sy, sz = lax.fori_loop(0, PPW // 16, mean_body, (zero16, zero16, zero16))
    sumv[pl.ds(0, 16)] = sx
    sumv[pl.ds(16, 16)] = sy
    sumv[pl.ds(32, 16)] = sz
    pltpu.sync_copy(sumv, shared.at[sid])
    plsc.subcore_barrier()
    w0 = (b % 4) * 4
    pltpu.sync_copy(shared.at[w0], sumv)
    s0x, s0y, s0z = sumv[pl.ds(0, 16)], sumv[pl.ds(16, 16)], sumv[pl.ds(32, 16)]
    pltpu.sync_copy(shared.at[w0 + 1], sumv)
    s1x, s1y, s1z = sumv[pl.ds(0, 16)], sumv[pl.ds(16, 16)], sumv[pl.ds(32, 16)]
    pltpu.sync_copy(shared.at[w0 + 2], sumv)
    s2x, s2y, s2z = sumv[pl.ds(0, 16)], sumv[pl.ds(16, 16)], sumv[pl.ds(32, 16)]
    pltpu.sync_copy(shared.at[w0 + 3], sumv)
    s3x, s3y, s3z = sumv[pl.ds(0, 16)], sumv[pl.ds(16, 16)], sumv[pl.ds(32, 16)]
    inv_n = 1.0 / N
    midx = jnp.sum((s0x + s1x) + (s2x + s3x)) * inv_n
    midy = jnp.sum((s0y + s1y) + (s2y + s3y)) * inv_n
    midz = jnp.sum((s0z + s1z) + (s2z + s3z)) * inv_n

    qx = plsc.load_gather(qbuf, [lane * 4 + 1])
    qy = plsc.load_gather(qbuf, [lane * 4 + 2])
    qz = plsc.load_gather(qbuf, [lane * 4 + 3])
    rinv = _rsqrt(qx * qx + qy * qy + qz * qz)
    w = jnp.full((16,), 0.707, jnp.float32)
    x = 0.707 * (qx * rinv)
    y = 0.707 * (qy * rinv)
    z = 0.707 * (qz * rinv)
    m00 = w * w + x * x - y * y - z * z
    m01 = 2.0 * (x * y - w * z)
    m02 = 2.0 * (x * z + w * y)
    m10 = 2.0 * (x * y + w * z)
    m11 = w * w - x * x + y * y - z * z
    m12 = 2.0 * (y * z - w * x)
    m20 = 2.0 * (x * z - w * y)
    m21 = 2.0 * (y * z + w * x)
    m22 = w * w - x * x - y * y + z * z
    t0 = -(m00 * midx + m01 * midy + m02 * midz)
    t1 = -(m10 * midx + m11 * midy + m12 * midz)
    t2 = -(m20 * midx + m21 * midy + m22 * midz)
    lane16 = lane * 16
    for k, vec in enumerate((m00, m01, m02, m10, m11, m12, m20, m21, m22,
                             t0, t1, t2)):
        plsc.store_scatter(coef_v, [lane16 + k], vec)

    base_vox = (b * G3).astype(jnp.int32)
    base_cpx = ((0 * B + b) * G3).astype(jnp.int32)
    base_cpy = ((1 * B + b) * G3).astype(jnp.int32)
    base_cpz = ((2 * B + b) * G3).astype(jnp.int32)
    wq = (wid % 4) * 4

    def chunk_fire(c, _):
        qb = wq + c // 4
        crow = coef_v[pl.ds(pl.multiple_of(qb * 16, 16), 16)]
        c00 = crow[0]
        c01 = crow[1]
        c02 = crow[2]
        c10 = crow[3]
        c11 = crow[4]
        c12 = crow[5]
        c20 = crow[6]
        c21 = crow[7]
        c22 = crow[8]
        d0 = crow[9]
        d1 = crow[10]
        d2 = crow[11]

        def phase_a(i, _):
            off = pl.multiple_of(c * CHUNK + i * 16, 16)
            vx = px[pl.ds(off, 16)]
            vy = py[pl.ds(off, 16)]
            vz = pz[pl.ds(off, 16)]
            rx = c00 * vx + c01 * vy + c02 * vz + d0
            ry = c10 * vx + c11 * vy + c12 * vz + d1
            rz = c20 * vx + c21 * vy + c22 * vz + d2

            def vceil(t):
                ti = t.astype(jnp.int32)
                tf = ti.astype(jnp.float32)
                return ti + jnp.where(t > tf, 1, 0).astype(jnp.int32)

            ix = vceil((rx + 0.5) * G - 0.5)
            iy = vceil((ry + 0.5) * G - 0.5)
            iz = vceil((rz + 0.5) * G - 0.5)
            ind = ix * (G * G) + iy * G + iz
            ind = jnp.minimum(jnp.maximum(ind, 0), G3 - 1)
            soff = pl.multiple_of(i * 16, 16)
            rbx[c, pl.ds(soff, 16)] = rx
            rby[c, pl.ds(soff, 16)] = ry
            rbz[c, pl.ds(soff, 16)] = rz
            idxv[c, pl.ds(soff, 16)] = ind + base_vox
            idx0[c, pl.ds(soff, 16)] = ind + base_cpx
            idx1[c, pl.ds(soff, 16)] = ind + base_cpy
            idx2[c, pl.ds(soff, 16)] = ind + base_cpz
            return 0

        lax.fori_loop(0, SUB, phase_a, 0)
        pltpu.async_copy(cp_hbm.at[idx0.at[c]], gx.at[c], sem)
        pltpu.async_copy(cp_hbm.at[idx1.at[c]], gy.at[c], sem)
        pltpu.async_copy(cp_hbm.at[idx2.at[c]], gz.at[c], sem)
        pltpu.async_copy(vox_hbm.at[idxv.at[c]], gv.at[c], sem)
        return 0

    lax.fori_loop(0, NCHUNK, chunk_fire, 0)

    def chunk_drain(c, acc):
        pltpu.make_async_copy(cp_hbm.at[idx0.at[c]], gx.at[c], sem).wait()
        pltpu.make_async_copy(cp_hbm.at[idx1.at[c]], gy.at[c], sem).wait()
        pltpu.make_async_copy(cp_hbm.at[idx2.at[c]], gz.at[c], sem).wait()
        pltpu.make_async_copy(vox_hbm.at[idxv.at[c]], gv.at[c], sem).wait()

        def phase_c(i, acc):
            soff = pl.multiple_of(i * 16, 16)
            m = 1.0 - gv[c, pl.ds(soff, 16)]
            dx = (rbx[c, pl.ds(soff, 16)] - gx[c, pl.ds(soff, 16)]) * m
            dy = (rby[c, pl.ds(soff, 16)] - gy[c, pl.ds(soff, 16)]) * m
            dz = (rbz[c, pl.ds(soff, 16)] - gz[c, pl.ds(soff, 16)]) * m
            return acc + (dx * dx + dy * dy + dz * dz)

        return lax.fori_loop(0, SUB, phase_c, acc)

    acc = lax.fori_loop(0, NCHUNK, chunk_drain, jnp.zeros((16,), jnp.float32))
    accv[pl.ds(0, 16)] = acc
    for r in range(1, 8):
        accv[pl.ds(r * 16, 16)] = zero16
    pltpu.sync_copy(accv, out_hbm.at[wid])


def _tc_reduce_body(x_ref, o_ref):
    o_ref[0, 0] = jnp.sum(x_ref[...]) * (1.0 / (B * N))


def kernel(voxel, points, closest_points, quads):
    pts_flat = points.transpose(2, 0, 1).reshape(3 * B * N)
    cp_flat = closest_points.transpose(2, 0, 1).reshape(3 * B * G3)
    vox_flat = voxel.reshape(B * G3)
    quads_flat = quads.reshape(B * Q * 4)

    mesh = plsc.VectorSubcoreMesh(core_axis_name="c", subcore_axis_name="s")
    partials = pl.kernel(
        _sc_body,
        out_type=jax.ShapeDtypeStruct((NW, 128), jnp.float32),
        mesh=mesh,
        compiler_params=pltpu.CompilerParams(
            use_tc_tiling_on_sc=False, needs_layout_passes=False),
        scratch_types=[
            pltpu.VMEM((PPW,), jnp.float32),
            pltpu.VMEM((PPW,), jnp.float32),
            pltpu.VMEM((PPW,), jnp.float32),
            pltpu.VMEM((Q * 4,), jnp.float32),
            pltpu.VMEM((Q * 16,), jnp.float32),
            pltpu.VMEM((48,), jnp.float32),
            pltpu.VMEM((NCHUNK, CHUNK), jnp.int32),
            pltpu.VMEM((NCHUNK, CHUNK), jnp.int32),
            pltpu.VMEM((NCHUNK, CHUNK), jnp.int32),
            pltpu.VMEM((NCHUNK, CHUNK), jnp.int32),
            pltpu.VMEM((NCHUNK, CHUNK), jnp.float32),
            pltpu.VMEM((NCHUNK, CHUNK), jnp.float32),
            pltpu.VMEM((NCHUNK, CHUNK), jnp.float32),
            pltpu.VMEM((NCHUNK, CHUNK), jnp.float32),
            pltpu.VMEM((NCHUNK, CHUNK), jnp.float32),
            pltpu.VMEM((NCHUNK, CHUNK), jnp.float32),
            pltpu.VMEM((NCHUNK, CHUNK), jnp.float32),
            pltpu.VMEM((128,), jnp.float32),
            pltpu.VMEM_SHARED((16, 48), jnp.float32),
            pltpu.SemaphoreType.DMA,
        ],
    )(pts_flat, cp_flat, vox_flat, quads_flat)

    total = pl.pallas_call(
        _tc_reduce_body,
        out_shape=jax.ShapeDtypeStruct((1, 1), jnp.float32),
        out_specs=pl.BlockSpec(memory_space=pltpu.SMEM),
    )(partials)
    return total[0, 0]
